# baseline (speedup 1.0000x reference)
import functools

import jax
import jax.numpy as jnp
from jax import lax
from jax.experimental import pallas as pl
from jax.experimental.pallas import tpu as pltpu
from jax.experimental.pallas import tpu_sc as plsc

B, C, H, W = 4, 3, 224, 224
K2 = 25
HW = H * W
NW = 32
ROWS_PER = H // NW
SLAB = ROWS_PER * W
GROUPS = W // 16
H_P, W_P = 231, 232
PHW = H_P * W_P
A = -0.75


def _cubic_coeffs(t):
    u = 1.0 - t
    c0 = (A * t) * u * u
    c3 = (A * u) * t * t
    c1 = u * ((1.0 + t) - (1.25 * t) * t)
    c2 = t * ((1.0 + u) - (1.25 * u) * u)
    return (c0, c1, c2, c3)


def _sc_body(pk_hbm, c2_hbm, off_hbm, wt_hbm, out_hbm,
             pk_v, c2_v, dx_v, dy_v, wt_v, acc_v, sems):
    cid = lax.axis_index("c")
    sid = lax.axis_index("s")
    wid = sid * 2 + cid
    bat = wid // 8
    sub = wid % 8

    pltpu.sync_copy(pk_hbm.at[pl.ds(bat * PHW, PHW)], pk_v)
    pltpu.sync_copy(c2_hbm.at[pl.ds(bat * PHW, PHW)], c2_v)

    zeros16 = jnp.zeros((16,), jnp.float32)

    def issue(k, base, buf):
        off_base = (bat * 2 * K2 + 2 * k) * HW + base
        wt_base = (bat * K2 + k) * HW + base
        pltpu.async_copy(off_hbm.at[pl.ds(off_base, SLAB)], dx_v.at[pl.ds(buf * SLAB, SLAB)], sems.at[buf])
        pltpu.async_copy(off_hbm.at[pl.ds(off_base + HW, SLAB)], dy_v.at[pl.ds(buf * SLAB, SLAB)], sems.at[buf])
        pltpu.async_copy(wt_hbm.at[pl.ds(wt_base, SLAB)], wt_v.at[pl.ds(buf * SLAB, SLAB)], sems.at[buf])

    def drain(base, buf):
        off0 = bat * 2 * K2 * HW + base
        pltpu.make_async_copy(off_hbm.at[pl.ds(off0, SLAB)], dx_v.at[pl.ds(buf * SLAB, SLAB)], sems.at[buf]).wait()
        pltpu.make_async_copy(off_hbm.at[pl.ds(off0, SLAB)], dy_v.at[pl.ds(buf * SLAB, SLAB)], sems.at[buf]).wait()
        pltpu.make_async_copy(off_hbm.at[pl.ds(off0, SLAB)], wt_v.at[pl.ds(buf * SLAB, SLAB)], sems.at[buf]).wait()

    def compute(row0, buf):
        def row_body(r, _):
            yf = (row0 + r).astype(jnp.float32)

            @plsc.parallel_loop(0, GROUPS, unroll=3)
            def g_body(g):
                pos = r * W + g * 16
                xi = lax.broadcasted_iota(jnp.int32, (16,), 0) + g * 16
                xf = xi.astype(jnp.float32)
                dx = dx_v[pl.ds(buf * SLAB + pos, 16)]
                dy = dy_v[pl.ds(buf * SLAB + pos, 16)]
                wt = wt_v[pl.ds(buf * SLAB + pos, 16)]
                sx4 = jnp.clip((xf + 4.0) + dx, 2.0, float(W + 5))
                sy4 = jnp.clip((yf + 4.0) + dy, 2.0, float(H + 5))
                ix0p = sx4.astype(jnp.int32)
                iy0p = sy4.astype(jnp.int32)
                tx = sx4 - ix0p.astype(jnp.float32)
                ty = sy4 - iy0p.astype(jnp.float32)
                addr00 = (iy0p - 2) * W_P + (ix0p - 2)
                cx = _cubic_coeffs(tx)
                cy = _cubic_coeffs(ty)
                cxp = [plsc.pack(c, c, format=plsc.PackFormat.INTERLEAVED)
                       for c in cx]
                cyp = [plsc.pack(c, c, format=plsc.PackFormat.INTERLEAVED)
                       for c in cy]
                s01 = jnp.zeros((32,), jnp.bfloat16)
                s2 = jnp.zeros((16,), jnp.float32)
                for i in range(4):
                    r01 = jnp.zeros((32,), jnp.bfloat16)
                    r2 = jnp.zeros((16,), jnp.float32)
                    for j in range(4):
                        addr = addr00 + (i * W_P + j)
                        p = plsc.load_gather(pk_v, [addr])
                        v2 = plsc.load_gather(c2_v, [addr])
                        v01 = plsc.bitcast(p, jnp.bfloat16)
                        r01 = r01 + cxp[j] * v01
                        r2 = r2 + cx[j] * v2
                    s01 = s01 + cyp[i] * r01
                    s2 = s2 + cy[i] * r2
                f0, f1 = plsc.unpack(s01, format=plsc.PackFormat.INTERLEAVED)
                plsc.addupdate(acc_v.at[pl.ds(pos, 16)], wt * f0)
                plsc.addupdate(acc_v.at[pl.ds(SLAB + pos, 16)], wt * f1)
                plsc.addupdate(acc_v.at[pl.ds(2 * SLAB + pos, 16)], wt * s2)

            return ()

        lax.fori_loop(0, ROWS_PER, row_body, ())

    def chunk_body(ch, _):
        row0 = sub * (4 * ROWS_PER) + ch * ROWS_PER
        base = row0 * W

        def zero_body(i, _):
            acc_v[pl.ds(i * 16, 16)] = zeros16
            return ()

        lax.fori_loop(0, 3 * SLAB // 16, zero_body, ())

        issue(0, base, 0)

        def k_body(kk, _):
            k0 = 2 * kk
            k1 = 2 * kk + 1
            drain(base, 0)
            issue(jnp.minimum(k1, K2 - 1), base, 1)
            compute(row0, 0)
            drain(base, 1)
            issue(jnp.minimum(k1 + 1, K2 - 1), base, 0)

            @pl.when(k1 < K2)
            def _():
                compute(row0, 1)
            return ()

        lax.fori_loop(0, (K2 + 1) // 2, k_body, ())
        drain(base, 0)

        for c in range(C):
            pltpu.sync_copy(acc_v.at[pl.ds(c * SLAB, SLAB)],
                            out_hbm.at[pl.ds((bat * C + c) * HW + base, SLAB)])
        return ()

    lax.fori_loop(0, 4, chunk_body, ())


_mesh = plsc.VectorSubcoreMesh(core_axis_name="c", subcore_axis_name="s")

_sc_kernel = functools.partial(
    pl.kernel,
    out_type=jax.ShapeDtypeStruct((B * C * HW,), jnp.float32),
    mesh=_mesh,
    scratch_types=[
        pltpu.VMEM((PHW,), jnp.int32),
        pltpu.VMEM((PHW,), jnp.float32),
        pltpu.VMEM((2 * SLAB,), jnp.float32),
        pltpu.VMEM((2 * SLAB,), jnp.float32),
        pltpu.VMEM((2 * SLAB,), jnp.float32),
        pltpu.VMEM((3 * SLAB,), jnp.float32),
        pltpu.SemaphoreType.DMA((2,)),
    ],
    compiler_params=pltpu.CompilerParams(needs_layout_passes=False),
)(_sc_body)


def kernel(image, offsets, weights):
    imgp = jnp.pad(image, ((0, 0), (0, 0), (3, 4), (3, 5)), mode="edge")
    ubf = lax.bitcast_convert_type(imgp.astype(jnp.bfloat16), jnp.uint16)
    ubf = ubf.astype(jnp.uint32)
    packed = lax.bitcast_convert_type(
        ubf[:, 0] | (ubf[:, 1] << 16), jnp.int32).reshape(B * PHW)
    c2 = imgp[:, 2].reshape(B * PHW)
    off = offsets.reshape(B * 2 * K2 * HW)
    wt = weights.reshape(B * K2 * HW)
    out = _sc_kernel(packed, c2, off, wt)
    return out.reshape(B, C, H, W)


# speedup vs baseline: 1.6570x; 1.6570x over previous
import functools

import jax
import jax.numpy as jnp
from jax import lax
from jax.experimental import pallas as pl
from jax.experimental.pallas import tpu as pltpu
from jax.experimental.pallas import tpu_sc as plsc

B, C, H, W = 4, 3, 224, 224
K2 = 25
HW = H * W
NW = 32
ROWS_PER = H // NW
SLAB = ROWS_PER * W
GROUPS = W // 16
H_P, W_P = 231, 232
PHW = H_P * W_P
A = -0.75


def _cubic_coeffs(t):
    u = 1.0 - t
    c0 = (A * t) * u * u
    c3 = (A * u) * t * t
    c1 = u * ((1.0 + t) - (1.25 * t) * t)
    c2 = t * ((1.0 + u) - (1.25 * u) * u)
    return (c0, c1, c2, c3)


def _sc_body(pk_hbm, c2_hbm, off_hbm, wt_hbm, out_hbm,
             pk_v, c2_v, dx_v, dy_v, wt_v, acc_v, sems):
    cid = lax.axis_index("c")
    sid = lax.axis_index("s")
    wid = sid * 2 + cid
    bat = wid // 8
    sub = wid % 8

    pltpu.sync_copy(pk_hbm.at[pl.ds(bat * PHW, PHW)], pk_v)
    pltpu.sync_copy(c2_hbm.at[pl.ds(bat * PHW, PHW)], c2_v)

    zeros16 = jnp.zeros((16,), jnp.float32)

    def issue(k, base, buf):
        off_base = (bat * 2 * K2 + 2 * k) * HW + base
        wt_base = (bat * K2 + k) * HW + base
        pltpu.async_copy(off_hbm.at[pl.ds(off_base, SLAB)], dx_v.at[pl.ds(buf * SLAB, SLAB)], sems.at[buf])
        pltpu.async_copy(off_hbm.at[pl.ds(off_base + HW, SLAB)], dy_v.at[pl.ds(buf * SLAB, SLAB)], sems.at[buf])
        pltpu.async_copy(wt_hbm.at[pl.ds(wt_base, SLAB)], wt_v.at[pl.ds(buf * SLAB, SLAB)], sems.at[buf])

    def drain(base, buf):
        off0 = bat * 2 * K2 * HW + base
        pltpu.make_async_copy(off_hbm.at[pl.ds(off0, SLAB)], dx_v.at[pl.ds(buf * SLAB, SLAB)], sems.at[buf]).wait()
        pltpu.make_async_copy(off_hbm.at[pl.ds(off0, SLAB)], dy_v.at[pl.ds(buf * SLAB, SLAB)], sems.at[buf]).wait()
        pltpu.make_async_copy(off_hbm.at[pl.ds(off0, SLAB)], wt_v.at[pl.ds(buf * SLAB, SLAB)], sems.at[buf]).wait()

    def compute(row0, buf):
        def row_body(r, _):
            yf = (row0 + r).astype(jnp.float32)

            @plsc.parallel_loop(0, GROUPS, unroll=2)
            def g_body(g):
                pos = r * W + g * 16
                xi = lax.broadcasted_iota(jnp.int32, (16,), 0) + g * 16
                xf = xi.astype(jnp.float32)
                dx = dx_v[pl.ds(buf * SLAB + pos, 16)]
                dy = dy_v[pl.ds(buf * SLAB + pos, 16)]
                wt = wt_v[pl.ds(buf * SLAB + pos, 16)]
                sx4 = jnp.clip((xf + 4.0) + dx, 2.0, float(W + 5))
                sy4 = jnp.clip((yf + 4.0) + dy, 2.0, float(H + 5))
                ix0p = sx4.astype(jnp.int32)
                iy0p = sy4.astype(jnp.int32)
                tx = sx4 - ix0p.astype(jnp.float32)
                ty = sy4 - iy0p.astype(jnp.float32)
                addr00 = (iy0p - 2) * W_P + (ix0p - 2)
                cx = _cubic_coeffs(tx)
                cy = _cubic_coeffs(ty)
                cxp = [plsc.pack(c, c, format=plsc.PackFormat.INTERLEAVED)
                       for c in cx]
                cyp = [plsc.pack(c, c, format=plsc.PackFormat.INTERLEAVED)
                       for c in cy]
                s01 = jnp.zeros((32,), jnp.bfloat16)
                s2 = jnp.zeros((16,), jnp.float32)
                for i in range(4):
                    r01 = jnp.zeros((32,), jnp.bfloat16)
                    r2 = jnp.zeros((16,), jnp.float32)
                    for j in range(4):
                        addr = addr00 + (i * W_P + j)
                        p = plsc.load_gather(pk_v, [addr])
                        v2 = plsc.load_gather(c2_v, [addr])
                        v01 = plsc.bitcast(p, jnp.bfloat16)
                        r01 = r01 + cxp[j] * v01
                        r2 = r2 + cx[j] * v2
                    s01 = s01 + cyp[i] * r01
                    s2 = s2 + cy[i] * r2
                f0, f1 = plsc.unpack(s01, format=plsc.PackFormat.INTERLEAVED)
                plsc.addupdate(acc_v.at[pl.ds(pos, 16)], wt * f0)
                plsc.addupdate(acc_v.at[pl.ds(SLAB + pos, 16)], wt * f1)
                plsc.addupdate(acc_v.at[pl.ds(2 * SLAB + pos, 16)], wt * s2)

            return ()

        lax.fori_loop(0, ROWS_PER, row_body, ())

    def chunk_body(ch, _):
        row0 = sub * (4 * ROWS_PER) + ch * ROWS_PER
        base = row0 * W

        def zero_body(i, _):
            acc_v[pl.ds(i * 16, 16)] = zeros16
            return ()

        lax.fori_loop(0, 3 * SLAB // 16, zero_body, ())

        issue(0, base, 0)

        def k_body(kk, _):
            k0 = 2 * kk
            k1 = 2 * kk + 1
            drain(base, 0)
            issue(jnp.minimum(k1, K2 - 1), base, 1)
            compute(row0, 0)
            drain(base, 1)
            issue(jnp.minimum(k1 + 1, K2 - 1), base, 0)

            @pl.when(k1 < K2)
            def _():
                compute(row0, 1)
            return ()

        lax.fori_loop(0, (K2 + 1) // 2, k_body, ())
        drain(base, 0)

        for c in range(C):
            pltpu.sync_copy(acc_v.at[pl.ds(c * SLAB, SLAB)],
                            out_hbm.at[pl.ds((bat * C + c) * HW + base, SLAB)])
        return ()

    lax.fori_loop(0, 4, chunk_body, ())


_mesh = plsc.VectorSubcoreMesh(core_axis_name="c", subcore_axis_name="s")

_sc_kernel = functools.partial(
    pl.kernel,
    out_type=jax.ShapeDtypeStruct((B * C * HW,), jnp.float32),
    mesh=_mesh,
    scratch_types=[
        pltpu.VMEM((PHW,), jnp.int32),
        pltpu.VMEM((PHW,), jnp.float32),
        pltpu.VMEM((2 * SLAB,), jnp.float32),
        pltpu.VMEM((2 * SLAB,), jnp.float32),
        pltpu.VMEM((2 * SLAB,), jnp.float32),
        pltpu.VMEM((3 * SLAB,), jnp.float32),
        pltpu.SemaphoreType.DMA((2,)),
    ],
    compiler_params=pltpu.CompilerParams(needs_layout_passes=False),
)(_sc_body)


def kernel(image, offsets, weights):
    imgp = jnp.pad(image, ((0, 0), (0, 0), (3, 4), (3, 5)), mode="edge")
    ubf = lax.bitcast_convert_type(imgp.astype(jnp.bfloat16), jnp.uint16)
    ubf = ubf.astype(jnp.uint32)
    packed = lax.bitcast_convert_type(
        ubf[:, 0] | (ubf[:, 1] << 16), jnp.int32).reshape(B * PHW)
    c2 = imgp[:, 2].reshape(B * PHW)
    off = offsets.reshape(B * 2 * K2 * HW)
    wt = weights.reshape(B * K2 * HW)
    out = _sc_kernel(packed, c2, off, wt)
    return out.reshape(B, C, H, W)
